# baseline (speedup 1.0000x reference)
import functools

import jax
import jax.numpy as jnp
from jax import lax
from jax.experimental import pallas as pl
from jax.experimental.pallas import tpu as pltpu
import jax.experimental.pallas.tpu_sc as plsc

N = 50000
E = 800000
F = 64
HALF = 32
NLAYERS = 3
NGRAPH = 50
NODES_PER_GRAPH = N // NGRAPH

NSC = 2
NTILE = 16
CH = 128
NCH = 392
EP_TILE = NCH * CH
EPAD = NTILE * EP_TILE
NBUF = 4
G = 28
NG = NCH // G
NACC = 50016
ROWS_PER_TILE = NACC // NTILE
TRASH = N

_f32 = jnp.float32


L = 2


def _prop_body(tab, rows, cols, zz, out, acc, rbuf, cbuf, gbuf, gsem, ssem, isem):
    cid = lax.axis_index("c")
    sid = lax.axis_index("s")
    w = cid * NTILE + sid

    pltpu.sync_copy(zz, acc.at[pl.ds(sid * ROWS_PER_TILE, ROWS_PER_TILE)])
    pltpu.async_copy(rows.at[w, pl.ds(0, G)], rbuf.at[0], isem.at[0, 0])
    pltpu.async_copy(cols.at[sid, pl.ds(0, G)], cbuf.at[0], isem.at[0, 1])
    plsc.subcore_barrier()

    def do_gather(pb, jn, with_wait):
        bn = jn % NBUF
        if with_wait:
            pltpu.make_async_copy(
                gbuf.at[bn], acc.at[cbuf.at[pb, jn - NBUF]], ssem.at[bn]).wait()
        pltpu.async_copy(tab.at[rbuf.at[pb, jn]], gbuf.at[bn], gsem.at[bn])

    def do_chunk(pb, j):
        b = j % NBUF
        pltpu.make_async_copy(tab.at[rbuf.at[pb, j]], gbuf.at[b], gsem.at[b]).wait()
        pltpu.async_copy(gbuf.at[b], acc.at[cbuf.at[pb, j]], ssem.at[b], add=True)

    def outer(g2, carry):
        for pb in range(2):
            g = g2 * 2 + pb
            base = g * G
            pltpu.make_async_copy(
                rows.at[w, pl.ds(base, G)], rbuf.at[pb], isem.at[pb, 0]).wait()
            pltpu.make_async_copy(
                cols.at[sid, pl.ds(base, G)], cbuf.at[pb], isem.at[pb, 1]).wait()

            @pl.when(g + 1 < NG)
            def _():
                nb = (g + 1) * G
                pltpu.async_copy(
                    rows.at[w, pl.ds(nb, G)], rbuf.at[1 - pb], isem.at[1 - pb, 0])
                pltpu.async_copy(
                    cols.at[sid, pl.ds(nb, G)], cbuf.at[1 - pb], isem.at[1 - pb, 1])

            for j in range(L):
                do_gather(pb, j, with_wait=False)
            for j in range(NBUF):
                do_gather(pb, j + L, with_wait=(j + L >= NBUF))
                do_chunk(pb, j)

            def chunk_grp(q, c2):
                for b4 in range(NBUF):
                    j = q * NBUF + b4

                    @pl.when(j + L < G)
                    def _():
                        do_gather(pb, j + L, with_wait=True)

                    do_chunk(pb, j)
                return c2

            lax.fori_loop(1, G // NBUF, chunk_grp, 0)

            for b in range(NBUF):
                jlast = G - NBUF + b
                pltpu.make_async_copy(
                    gbuf.at[jlast % NBUF], acc.at[cbuf.at[pb, jlast]],
                    ssem.at[jlast % NBUF]).wait()
        return carry

    lax.fori_loop(0, NG // 2, outer, 0)

    plsc.subcore_barrier()
    base = sid * ROWS_PER_TILE
    pltpu.sync_copy(
        acc.at[pl.ds(base, ROWS_PER_TILE)],
        out.at[pl.ds(cid * NACC + base, ROWS_PER_TILE)],
    )


@functools.cache
def _make_prop():
    mesh = plsc.VectorSubcoreMesh(
        core_axis_name="c", subcore_axis_name="s", num_cores=NSC,
        num_subcores=NTILE)
    return pl.kernel(
        _prop_body,
        out_type=jax.ShapeDtypeStruct((NSC * NACC, HALF), _f32),
        mesh=mesh,
        scratch_types=[
            pltpu.VMEM_SHARED((NACC, HALF), _f32),
            pltpu.VMEM((2, G, CH), jnp.int32),
            pltpu.VMEM((2, G, CH), jnp.int32),
            pltpu.VMEM((NBUF, CH, HALF), _f32),
            pltpu.SemaphoreType.DMA((NBUF,)),
            pltpu.SemaphoreType.DMA((NBUF,)),
            pltpu.SemaphoreType.DMA((2, 2)),
        ],
        compiler_params=pltpu.CompilerParams(use_tc_tiling_on_sc=False),
    )


def _propagate(tab_flat, rows2, cols, zz):
    out = _make_prop()(tab_flat, rows2, cols, zz)
    return out.reshape(NSC, NACC, HALF)


_BT = 2000
_NB = N // _BT


def _relu(v):
    return jnp.maximum(v, 0.0)


def _p0_body(x_ref, wi_ref, we1_ref, cur_ref, hp_ref):
    xb = x_ref[...]
    cur = _relu(jnp.dot(xb, wi_ref[...], preferred_element_type=_f32))
    cur_ref[0] = cur[:, :HALF]
    cur_ref[1] = cur[:, HALF:]
    h = _relu(jnp.dot(xb, we1_ref[...], preferred_element_type=_f32))
    hp_ref[0] = h[:, :HALF]
    hp_ref[1] = jnp.concatenate(
        [h[:, HALF:], jnp.ones((xb.shape[0], 1), _f32)], axis=1)


def _p0(x, w_init, w_e1):
    return pl.pallas_call(
        _p0_body,
        grid=(_NB,),
        in_specs=[
            pl.BlockSpec((_BT, 4), lambda i: (i, 0)),
            pl.BlockSpec((4, F), lambda i: (0, 0)),
            pl.BlockSpec((4, F - 1), lambda i: (0, 0)),
        ],
        out_specs=[
            pl.BlockSpec((2, _BT, HALF), lambda i: (0, i, 0)),
            pl.BlockSpec((2, _BT, HALF), lambda i: (0, i, 0)),
        ],
        out_shape=[jax.ShapeDtypeStruct((2, N, HALF), _f32)] * 2,
    )(x, w_init, w_e1)


def _dmax_body(agg1_ref, dm_ref):
    i = pl.program_id(0)
    bm = jnp.max(agg1_ref[0][:, HALF - 1:HALF], keepdims=True)

    @pl.when(i == 0)
    def _():
        dm_ref[...] = bm

    @pl.when(i > 0)
    def _():
        dm_ref[...] = jnp.maximum(dm_ref[...], bm)


def _dmax(aggh):
    return pl.pallas_call(
        _dmax_body,
        grid=(_NB,),
        in_specs=[pl.BlockSpec((1, _BT, HALF), lambda i: (1, i, 0))],
        out_specs=pl.BlockSpec((1, 1), lambda i: (0, 0)),
        out_shape=jax.ShapeDtypeStruct((1, 1), _f32),
    )(aggh)


def _p2_body(agg_ref, we2_ref, dm_ref, ee_ref, dinv_ref):
    deg = agg_ref[1][:, HALF - 1:HALF]
    dinv = deg ** -1.0
    a0 = agg_ref[0]
    a1 = agg_ref[1]
    y = jnp.concatenate(
        [dinv * a0, dinv * a1[:, :HALF - 1], deg / dm_ref[...]], axis=1)
    ee_ref[...] = _relu(jnp.dot(y, we2_ref[...], preferred_element_type=_f32))
    dinv_ref[...] = jnp.broadcast_to(dinv, (dinv.shape[0], HALF))


def _p2(aggh, w_e2, dmax):
    return pl.pallas_call(
        _p2_body,
        grid=(_NB,),
        in_specs=[
            pl.BlockSpec((2, _BT, HALF), lambda i: (0, i, 0)),
            pl.BlockSpec((F, F), lambda i: (0, 0)),
            pl.BlockSpec((1, 1), lambda i: (0, 0)),
        ],
        out_specs=[
            pl.BlockSpec((_BT, F), lambda i: (i, 0)),
            pl.BlockSpec((_BT, HALF), lambda i: (i, 0)),
        ],
        out_shape=[
            jax.ShapeDtypeStruct((N, F), _f32),
            jax.ShapeDtypeStruct((N, HALF), _f32),
        ],
    )(aggh, w_e2, dmax)


def _p3_body(agg_ref, cur_ref, ee_ref, dinv_ref, wm_ref, wu_ref, new_ref):
    dinv = dinv_ref[...]
    xm = jnp.concatenate(
        [dinv * agg_ref[0], dinv * agg_ref[1], ee_ref[...]], axis=1)
    msg = _relu(jnp.dot(xm, wm_ref[...], preferred_element_type=_f32))
    xu = jnp.concatenate([cur_ref[0], cur_ref[1], msg], axis=1)
    new = _relu(jnp.dot(xu, wu_ref[...], preferred_element_type=_f32))
    new_ref[0] = new[:, :HALF]
    new_ref[1] = new[:, HALF:]


def _p3(agg, cur_tab, ee, dinv32, wm, wu):
    return pl.pallas_call(
        _p3_body,
        grid=(_NB,),
        in_specs=[
            pl.BlockSpec((2, _BT, HALF), lambda i: (0, i, 0)),
            pl.BlockSpec((2, _BT, HALF), lambda i: (0, i, 0)),
            pl.BlockSpec((_BT, F), lambda i: (i, 0)),
            pl.BlockSpec((_BT, HALF), lambda i: (i, 0)),
            pl.BlockSpec((2 * F, F), lambda i: (0, 0)),
            pl.BlockSpec((2 * F, F), lambda i: (0, 0)),
        ],
        out_specs=pl.BlockSpec((2, _BT, HALF), lambda i: (0, i, 0)),
        out_shape=jax.ShapeDtypeStruct((2, N, HALF), _f32),
    )(agg, cur_tab, ee, dinv32, wm, wu)


def _pool_body(cur_ref, pooled_ref):
    k = jnp.float32(NODES_PER_GRAPH)
    s0 = jnp.sum(cur_ref[0], axis=0, keepdims=True) / k
    s1 = jnp.sum(cur_ref[1], axis=0, keepdims=True) / k
    pooled_ref[...] = jnp.concatenate([s0, s1], axis=1).reshape(1, 1, F)


def _pool(cur_tab):
    return pl.pallas_call(
        _pool_body,
        grid=(NGRAPH,),
        in_specs=[pl.BlockSpec((2, NODES_PER_GRAPH, HALF), lambda i: (0, i, 0))],
        out_specs=pl.BlockSpec((1, 1, F), lambda i: (i, 0, 0)),
        out_shape=jax.ShapeDtypeStruct((NGRAPH, 1, F), _f32),
    )(cur_tab)


def _ro_body(pooled_ref, cur_ref, wp_ref, wro_ref, bro_ref, out_ref):
    fp = jnp.dot(pooled_ref[0], wp_ref[...], preferred_element_type=_f32)
    feats = jnp.concatenate(
        [jnp.broadcast_to(_relu(fp), (NODES_PER_GRAPH, F)),
         _relu(jnp.concatenate([cur_ref[0], cur_ref[1]], axis=1))], axis=1)
    out_ref[...] = (
        jnp.dot(feats, wro_ref[...], preferred_element_type=_f32) + bro_ref[...])


def _ro(pooled, cur_tab, w_pool, w_ro, bro):
    return pl.pallas_call(
        _ro_body,
        grid=(NGRAPH,),
        in_specs=[
            pl.BlockSpec((1, 1, F), lambda i: (i, 0, 0)),
            pl.BlockSpec((2, NODES_PER_GRAPH, HALF), lambda i: (0, i, 0)),
            pl.BlockSpec((F, F), lambda i: (0, 0)),
            pl.BlockSpec((2 * F, 1), lambda i: (0, 0)),
            pl.BlockSpec((1, 1), lambda i: (0, 0)),
        ],
        out_specs=pl.BlockSpec((NODES_PER_GRAPH, 1), lambda i: (i, 0)),
        out_shape=jax.ShapeDtypeStruct((N, 1), _f32),
    )(pooled, cur_tab, w_pool, w_ro, bro)


def kernel(x, edge_index, batch, batch_size, W_init, W_e1, W_e2, W_msg,
           W_upd, W_pool, W_ro, b_ro):
    del batch, batch_size
    pad = EPAD - E
    rows = jnp.concatenate(
        [edge_index[0], jnp.zeros((pad,), jnp.int32)]).reshape(NTILE, NCH, CH)
    rows2 = jnp.concatenate([rows, rows + N], axis=0)
    cols = jnp.concatenate(
        [edge_index[1], jnp.full((pad,), TRASH, jnp.int32)]).reshape(NTILE, NCH, CH)
    zz = jnp.zeros((ROWS_PER_TILE, HALF), _f32)

    cur_tab, hp_tab = _p0(x, W_init, W_e1)

    aggh = _propagate(hp_tab.reshape(2 * N, HALF), rows2, cols, zz)
    ee, dinv32 = _p2(aggh, W_e2, _dmax(aggh))

    bro = b_ro.reshape(1, 1)

    for i in range(NLAYERS):
        agg = _propagate(cur_tab.reshape(2 * N, HALF), rows2, cols, zz)
        cur_tab = _p3(agg, cur_tab, ee, dinv32, W_msg[i], W_upd[i])

    pooled = _pool(cur_tab)
    return _ro(pooled, cur_tab, W_pool, W_ro, bro)


# speedup vs baseline: 1.0293x; 1.0293x over previous
import functools

import jax
import jax.numpy as jnp
from jax import lax
from jax.experimental import pallas as pl
from jax.experimental.pallas import tpu as pltpu
import jax.experimental.pallas.tpu_sc as plsc

N = 50000
E = 800000
F = 64
HALF = 32
NLAYERS = 3
NGRAPH = 50
NODES_PER_GRAPH = N // NGRAPH

NSC = 2
NTILE = 16
CH = 128
NCH = 392
EP_TILE = NCH * CH
EPAD = NTILE * EP_TILE
NBUF = 4
G = 28
NG = NCH // G
NACC = 50016
ROWS_PER_TILE = NACC // NTILE
TRASH = N

_f32 = jnp.float32


def _prop_body(tab, rows, cols, zz, out, acc, rbuf, cbuf, gbuf, gsem, isem):
    cid = lax.axis_index("c")
    sid = lax.axis_index("s")
    w = cid * NTILE + sid

    pltpu.sync_copy(zz, acc.at[pl.ds(sid * ROWS_PER_TILE, ROWS_PER_TILE)])
    pltpu.async_copy(rows.at[w, pl.ds(0, G)], rbuf.at[0], isem.at[0, 0])
    pltpu.async_copy(cols.at[sid, pl.ds(0, G)], cbuf.at[0], isem.at[0, 1])
    plsc.subcore_barrier()

    def outer(g2, carry):
        for pb in range(2):
            g = g2 * 2 + pb
            base = g * G
            pltpu.make_async_copy(
                rows.at[w, pl.ds(base, G)], rbuf.at[pb], isem.at[pb, 0]).wait()
            pltpu.make_async_copy(
                cols.at[sid, pl.ds(base, G)], cbuf.at[pb], isem.at[pb, 1]).wait()

            @pl.when(g + 1 < NG)
            def _():
                nb = (g + 1) * G
                pltpu.async_copy(
                    rows.at[w, pl.ds(nb, G)], rbuf.at[1 - pb], isem.at[1 - pb, 0])
                pltpu.async_copy(
                    cols.at[sid, pl.ds(nb, G)], cbuf.at[1 - pb], isem.at[1 - pb, 1])

            for b in range(NBUF):
                pltpu.async_copy(tab.at[rbuf.at[pb, b]], gbuf.at[b], gsem.at[b])

            def chunk_grp(q, c2):
                for b in range(NBUF):
                    j = q * NBUF + b
                    pltpu.make_async_copy(
                        tab.at[rbuf.at[pb, j]], gbuf.at[b], gsem.at[b]).wait()
                    pltpu.sync_copy(gbuf.at[b], acc.at[cbuf.at[pb, j]], add=True)

                    @pl.when(j + NBUF < G)
                    def _():
                        pltpu.async_copy(
                            tab.at[rbuf.at[pb, j + NBUF]], gbuf.at[b], gsem.at[b])

                return c2

            lax.fori_loop(0, G // NBUF, chunk_grp, 0)
        return carry

    lax.fori_loop(0, NG // 2, outer, 0)

    plsc.subcore_barrier()
    base = sid * ROWS_PER_TILE
    pltpu.sync_copy(
        acc.at[pl.ds(base, ROWS_PER_TILE)],
        out.at[pl.ds(cid * NACC + base, ROWS_PER_TILE)],
    )


@functools.cache
def _make_prop():
    mesh = plsc.VectorSubcoreMesh(
        core_axis_name="c", subcore_axis_name="s", num_cores=NSC,
        num_subcores=NTILE)
    return pl.kernel(
        _prop_body,
        out_type=jax.ShapeDtypeStruct((NSC * NACC, HALF), _f32),
        mesh=mesh,
        scratch_types=[
            pltpu.VMEM_SHARED((NACC, HALF), _f32),
            pltpu.VMEM((2, G, CH), jnp.int32),
            pltpu.VMEM((2, G, CH), jnp.int32),
            pltpu.VMEM((NBUF, CH, HALF), _f32),
            pltpu.SemaphoreType.DMA((NBUF,)),
            pltpu.SemaphoreType.DMA((2, 2)),
        ],
        compiler_params=pltpu.CompilerParams(use_tc_tiling_on_sc=False),
    )


def _propagate(tab_flat, rows2, cols, zz):
    out = _make_prop()(tab_flat, rows2, cols, zz)
    return out.reshape(NSC, NACC, HALF)


_BT = 2000
_NB = N // _BT


def _relu(v):
    return jnp.maximum(v, 0.0)


def _p0_body(x_ref, wi_ref, we1_ref, cur_ref, hp_ref):
    xb = x_ref[...]
    cur = _relu(jnp.dot(xb, wi_ref[...], preferred_element_type=_f32))
    cur_ref[0] = cur[:, :HALF]
    cur_ref[1] = cur[:, HALF:]
    h = _relu(jnp.dot(xb, we1_ref[...], preferred_element_type=_f32))
    hp_ref[0] = h[:, :HALF]
    hp_ref[1] = jnp.concatenate(
        [h[:, HALF:], jnp.ones((xb.shape[0], 1), _f32)], axis=1)


def _p0(x, w_init, w_e1):
    return pl.pallas_call(
        _p0_body,
        grid=(_NB,),
        in_specs=[
            pl.BlockSpec((_BT, 4), lambda i: (i, 0)),
            pl.BlockSpec((4, F), lambda i: (0, 0)),
            pl.BlockSpec((4, F - 1), lambda i: (0, 0)),
        ],
        out_specs=[
            pl.BlockSpec((2, _BT, HALF), lambda i: (0, i, 0)),
            pl.BlockSpec((2, _BT, HALF), lambda i: (0, i, 0)),
        ],
        out_shape=[jax.ShapeDtypeStruct((2, N, HALF), _f32)] * 2,
    )(x, w_init, w_e1)


def _dmax_body(agg1_ref, dm_ref):
    i = pl.program_id(0)
    bm = jnp.max(agg1_ref[0][:, HALF - 1:HALF], keepdims=True)

    @pl.when(i == 0)
    def _():
        dm_ref[...] = bm

    @pl.when(i > 0)
    def _():
        dm_ref[...] = jnp.maximum(dm_ref[...], bm)


def _dmax(aggh):
    return pl.pallas_call(
        _dmax_body,
        grid=(_NB,),
        in_specs=[pl.BlockSpec((1, _BT, HALF), lambda i: (1, i, 0))],
        out_specs=pl.BlockSpec((1, 1), lambda i: (0, 0)),
        out_shape=jax.ShapeDtypeStruct((1, 1), _f32),
    )(aggh)


def _p2_body(agg_ref, we2_ref, dm_ref, ee_ref, dinv_ref):
    deg = agg_ref[1][:, HALF - 1:HALF]
    dinv = deg ** -1.0
    a0 = agg_ref[0]
    a1 = agg_ref[1]
    y = jnp.concatenate(
        [dinv * a0, dinv * a1[:, :HALF - 1], deg / dm_ref[...]], axis=1)
    ee_ref[...] = _relu(jnp.dot(y, we2_ref[...], preferred_element_type=_f32))
    dinv_ref[...] = jnp.broadcast_to(dinv, (dinv.shape[0], HALF))


def _p2(aggh, w_e2, dmax):
    return pl.pallas_call(
        _p2_body,
        grid=(_NB,),
        in_specs=[
            pl.BlockSpec((2, _BT, HALF), lambda i: (0, i, 0)),
            pl.BlockSpec((F, F), lambda i: (0, 0)),
            pl.BlockSpec((1, 1), lambda i: (0, 0)),
        ],
        out_specs=[
            pl.BlockSpec((_BT, F), lambda i: (i, 0)),
            pl.BlockSpec((_BT, HALF), lambda i: (i, 0)),
        ],
        out_shape=[
            jax.ShapeDtypeStruct((N, F), _f32),
            jax.ShapeDtypeStruct((N, HALF), _f32),
        ],
    )(aggh, w_e2, dmax)


def _p3_body(agg_ref, cur_ref, ee_ref, dinv_ref, wm_ref, wu_ref, new_ref):
    dinv = dinv_ref[...]
    xm = jnp.concatenate(
        [dinv * agg_ref[0], dinv * agg_ref[1], ee_ref[...]], axis=1)
    msg = _relu(jnp.dot(xm, wm_ref[...], preferred_element_type=_f32))
    xu = jnp.concatenate([cur_ref[0], cur_ref[1], msg], axis=1)
    new = _relu(jnp.dot(xu, wu_ref[...], preferred_element_type=_f32))
    new_ref[0] = new[:, :HALF]
    new_ref[1] = new[:, HALF:]


def _p3(agg, cur_tab, ee, dinv32, wm, wu):
    return pl.pallas_call(
        _p3_body,
        grid=(_NB,),
        in_specs=[
            pl.BlockSpec((2, _BT, HALF), lambda i: (0, i, 0)),
            pl.BlockSpec((2, _BT, HALF), lambda i: (0, i, 0)),
            pl.BlockSpec((_BT, F), lambda i: (i, 0)),
            pl.BlockSpec((_BT, HALF), lambda i: (i, 0)),
            pl.BlockSpec((2 * F, F), lambda i: (0, 0)),
            pl.BlockSpec((2 * F, F), lambda i: (0, 0)),
        ],
        out_specs=pl.BlockSpec((2, _BT, HALF), lambda i: (0, i, 0)),
        out_shape=jax.ShapeDtypeStruct((2, N, HALF), _f32),
    )(agg, cur_tab, ee, dinv32, wm, wu)


def _pool_body(cur_ref, pooled_ref):
    k = jnp.float32(NODES_PER_GRAPH)
    s0 = jnp.sum(cur_ref[0], axis=0, keepdims=True) / k
    s1 = jnp.sum(cur_ref[1], axis=0, keepdims=True) / k
    pooled_ref[...] = jnp.concatenate([s0, s1], axis=1).reshape(1, 1, F)


def _pool(cur_tab):
    return pl.pallas_call(
        _pool_body,
        grid=(NGRAPH,),
        in_specs=[pl.BlockSpec((2, NODES_PER_GRAPH, HALF), lambda i: (0, i, 0))],
        out_specs=pl.BlockSpec((1, 1, F), lambda i: (i, 0, 0)),
        out_shape=jax.ShapeDtypeStruct((NGRAPH, 1, F), _f32),
    )(cur_tab)


def _ro_body(pooled_ref, cur_ref, wp_ref, wro_ref, bro_ref, out_ref):
    fp = jnp.dot(pooled_ref[0], wp_ref[...], preferred_element_type=_f32)
    feats = jnp.concatenate(
        [jnp.broadcast_to(_relu(fp), (NODES_PER_GRAPH, F)),
         _relu(jnp.concatenate([cur_ref[0], cur_ref[1]], axis=1))], axis=1)
    out_ref[...] = (
        jnp.dot(feats, wro_ref[...], preferred_element_type=_f32) + bro_ref[...])


def _ro(pooled, cur_tab, w_pool, w_ro, bro):
    return pl.pallas_call(
        _ro_body,
        grid=(NGRAPH,),
        in_specs=[
            pl.BlockSpec((1, 1, F), lambda i: (i, 0, 0)),
            pl.BlockSpec((2, NODES_PER_GRAPH, HALF), lambda i: (0, i, 0)),
            pl.BlockSpec((F, F), lambda i: (0, 0)),
            pl.BlockSpec((2 * F, 1), lambda i: (0, 0)),
            pl.BlockSpec((1, 1), lambda i: (0, 0)),
        ],
        out_specs=pl.BlockSpec((NODES_PER_GRAPH, 1), lambda i: (i, 0)),
        out_shape=jax.ShapeDtypeStruct((N, 1), _f32),
    )(pooled, cur_tab, w_pool, w_ro, bro)


def kernel(x, edge_index, batch, batch_size, W_init, W_e1, W_e2, W_msg,
           W_upd, W_pool, W_ro, b_ro):
    del batch, batch_size
    pad = EPAD - E
    rows = jnp.concatenate(
        [edge_index[0], jnp.zeros((pad,), jnp.int32)]).reshape(NTILE, NCH, CH)
    rows2 = jnp.concatenate([rows, rows + N], axis=0)
    cols = jnp.concatenate(
        [edge_index[1], jnp.full((pad,), TRASH, jnp.int32)]).reshape(NTILE, NCH, CH)
    zz = jnp.zeros((ROWS_PER_TILE, HALF), _f32)

    cur_tab, hp_tab = _p0(x, W_init, W_e1)

    aggh = _propagate(hp_tab.reshape(2 * N, HALF), rows2, cols, zz)
    ee, dinv32 = _p2(aggh, W_e2, _dmax(aggh))

    bro = b_ro.reshape(1, 1)

    for i in range(NLAYERS):
        agg = _propagate(cur_tab.reshape(2 * N, HALF), rows2, cols, zz)
        cur_tab = _p3(agg, cur_tab, ee, dinv32, W_msg[i], W_upd[i])

    pooled = _pool(cur_tab)
    return _ro(pooled, cur_tab, W_pool, W_ro, bro)


# speedup vs baseline: 1.0561x; 1.0260x over previous
import functools

import jax
import jax.numpy as jnp
from jax import lax
from jax.experimental import pallas as pl
from jax.experimental.pallas import tpu as pltpu
import jax.experimental.pallas.tpu_sc as plsc

N = 50000
E = 800000
F = 64
HALF = 32
NLAYERS = 3
NGRAPH = 50
NODES_PER_GRAPH = N // NGRAPH

NSC = 2
NTILE = 16
CH = 128
NCH = 392
EP_TILE = NCH * CH
EPAD = NTILE * EP_TILE
NBUF = 4
G = 28
NG = NCH // G
NACC = 50016
ROWS_PER_TILE = NACC // NTILE
TRASH = N

_f32 = jnp.float32


def _prop_body(tab, rows, cols, zz, out, acc, rbuf, cbuf, gbuf, gsem, isem):
    cid = lax.axis_index("c")
    sid = lax.axis_index("s")
    w = cid * NTILE + sid

    pltpu.sync_copy(zz, acc.at[pl.ds(sid * ROWS_PER_TILE, ROWS_PER_TILE)])
    pltpu.async_copy(rows.at[w, pl.ds(0, G)], rbuf.at[0], isem.at[0, 0])
    pltpu.async_copy(cols.at[sid, pl.ds(0, G)], cbuf.at[0], isem.at[0, 1])
    plsc.subcore_barrier()

    def outer(g2, carry):
        for pb in range(2):
            g = g2 * 2 + pb
            base = g * G
            pltpu.make_async_copy(
                rows.at[w, pl.ds(base, G)], rbuf.at[pb], isem.at[pb, 0]).wait()
            pltpu.make_async_copy(
                cols.at[sid, pl.ds(base, G)], cbuf.at[pb], isem.at[pb, 1]).wait()

            @pl.when(g + 1 < NG)
            def _():
                nb = (g + 1) * G
                pltpu.async_copy(
                    rows.at[w, pl.ds(nb, G)], rbuf.at[1 - pb], isem.at[1 - pb, 0])
                pltpu.async_copy(
                    cols.at[sid, pl.ds(nb, G)], cbuf.at[1 - pb], isem.at[1 - pb, 1])

            for b in range(NBUF):
                pltpu.async_copy(tab.at[rbuf.at[pb, b]], gbuf.at[b], gsem.at[b])

            def chunk_grp(q, c2):
                for b in range(NBUF):
                    j = q * NBUF + b
                    pltpu.make_async_copy(
                        tab.at[rbuf.at[pb, j]], gbuf.at[b], gsem.at[b]).wait()
                    pltpu.sync_copy(gbuf.at[b], acc.at[cbuf.at[pb, j]], add=True)

                    @pl.when(j + NBUF < G)
                    def _():
                        pltpu.async_copy(
                            tab.at[rbuf.at[pb, j + NBUF]], gbuf.at[b], gsem.at[b])

                return c2

            lax.fori_loop(0, G // NBUF, chunk_grp, 0)
        return carry

    lax.fori_loop(0, NG // 2, outer, 0)

    plsc.subcore_barrier()
    base = sid * ROWS_PER_TILE
    pltpu.sync_copy(
        acc.at[pl.ds(base, ROWS_PER_TILE)],
        out.at[pl.ds(cid * NACC + base, ROWS_PER_TILE)],
    )


@functools.cache
def _make_prop():
    mesh = plsc.VectorSubcoreMesh(
        core_axis_name="c", subcore_axis_name="s", num_cores=NSC,
        num_subcores=NTILE)
    return pl.kernel(
        _prop_body,
        out_type=jax.ShapeDtypeStruct((NSC * NACC, HALF), _f32),
        mesh=mesh,
        scratch_types=[
            pltpu.VMEM_SHARED((NACC, HALF), _f32),
            pltpu.VMEM((2, G, CH), jnp.int32),
            pltpu.VMEM((2, G, CH), jnp.int32),
            pltpu.VMEM((NBUF, CH, HALF), _f32),
            pltpu.SemaphoreType.DMA((NBUF,)),
            pltpu.SemaphoreType.DMA((2, 2)),
        ],
        compiler_params=pltpu.CompilerParams(use_tc_tiling_on_sc=False),
    )


def _propagate(tab_flat, rows2, cols, zz):
    out = _make_prop()(tab_flat, rows2, cols, zz)
    return out.reshape(NSC, NACC, HALF)


_BT = 2000
_NB = N // _BT


def _relu(v):
    return jnp.maximum(v, 0.0)


def _p0_body(x_ref, wi_ref, we1_ref, cur_ref, hp_ref):
    xb = x_ref[...]
    cur = _relu(jnp.dot(xb, wi_ref[...], preferred_element_type=_f32))
    cur_ref[0] = cur[:, :HALF]
    cur_ref[1] = cur[:, HALF:]
    h = _relu(jnp.dot(xb, we1_ref[...], preferred_element_type=_f32))
    hp_ref[0] = h[:, :HALF]
    hp_ref[1] = jnp.concatenate(
        [h[:, HALF:], jnp.ones((xb.shape[0], 1), _f32)], axis=1)


def _p0(x, w_init, w_e1):
    return pl.pallas_call(
        _p0_body,
        grid=(_NB,),
        in_specs=[
            pl.BlockSpec((_BT, 4), lambda i: (i, 0)),
            pl.BlockSpec((4, F), lambda i: (0, 0)),
            pl.BlockSpec((4, F - 1), lambda i: (0, 0)),
        ],
        out_specs=[
            pl.BlockSpec((2, _BT, HALF), lambda i: (0, i, 0)),
            pl.BlockSpec((2, _BT, HALF), lambda i: (0, i, 0)),
        ],
        out_shape=[jax.ShapeDtypeStruct((2, N, HALF), _f32)] * 2,
    )(x, w_init, w_e1)


def _dmax_body(agg1_ref, dm_ref):
    i = pl.program_id(0)
    bm = jnp.max(agg1_ref[0][:, HALF - 1:HALF], keepdims=True)

    @pl.when(i == 0)
    def _():
        dm_ref[...] = bm

    @pl.when(i > 0)
    def _():
        dm_ref[...] = jnp.maximum(dm_ref[...], bm)


def _dmax(aggh):
    return pl.pallas_call(
        _dmax_body,
        grid=(_NB,),
        in_specs=[pl.BlockSpec((1, _BT, HALF), lambda i: (1, i, 0))],
        out_specs=pl.BlockSpec((1, 1), lambda i: (0, 0)),
        out_shape=jax.ShapeDtypeStruct((1, 1), _f32),
    )(aggh)


def _p2_body(agg_ref, we2_ref, dm_ref, ee_ref, dinv_ref):
    deg = agg_ref[1][:, HALF - 1:HALF]
    dinv = deg ** -1.0
    a0 = agg_ref[0]
    a1 = agg_ref[1]
    y = jnp.concatenate(
        [dinv * a0, dinv * a1[:, :HALF - 1], deg / dm_ref[...]], axis=1)
    ee_ref[...] = _relu(jnp.dot(y, we2_ref[...], preferred_element_type=_f32))
    dinv_ref[...] = jnp.broadcast_to(dinv, (dinv.shape[0], HALF))


def _p2(aggh, w_e2, dmax):
    return pl.pallas_call(
        _p2_body,
        grid=(_NB,),
        in_specs=[
            pl.BlockSpec((2, _BT, HALF), lambda i: (0, i, 0)),
            pl.BlockSpec((F, F), lambda i: (0, 0)),
            pl.BlockSpec((1, 1), lambda i: (0, 0)),
        ],
        out_specs=[
            pl.BlockSpec((_BT, F), lambda i: (i, 0)),
            pl.BlockSpec((_BT, HALF), lambda i: (i, 0)),
        ],
        out_shape=[
            jax.ShapeDtypeStruct((N, F), _f32),
            jax.ShapeDtypeStruct((N, HALF), _f32),
        ],
    )(aggh, w_e2, dmax)


def _p3_body(agg_ref, cur_ref, ee_ref, dinv_ref, wm_ref, wu_ref, new_ref):
    dinv = dinv_ref[...]
    xm = jnp.concatenate(
        [dinv * agg_ref[0], dinv * agg_ref[1], ee_ref[...]], axis=1)
    msg = _relu(jnp.dot(xm, wm_ref[...], preferred_element_type=_f32))
    xu = jnp.concatenate([cur_ref[0], cur_ref[1], msg], axis=1)
    new = _relu(jnp.dot(xu, wu_ref[...], preferred_element_type=_f32))
    new_ref[0] = new[:, :HALF]
    new_ref[1] = new[:, HALF:]


def _p3(agg, cur_tab, ee, dinv32, wm, wu):
    return pl.pallas_call(
        _p3_body,
        grid=(_NB,),
        in_specs=[
            pl.BlockSpec((2, _BT, HALF), lambda i: (0, i, 0)),
            pl.BlockSpec((2, _BT, HALF), lambda i: (0, i, 0)),
            pl.BlockSpec((_BT, F), lambda i: (i, 0)),
            pl.BlockSpec((_BT, HALF), lambda i: (i, 0)),
            pl.BlockSpec((2 * F, F), lambda i: (0, 0)),
            pl.BlockSpec((2 * F, F), lambda i: (0, 0)),
        ],
        out_specs=pl.BlockSpec((2, _BT, HALF), lambda i: (0, i, 0)),
        out_shape=jax.ShapeDtypeStruct((2, N, HALF), _f32),
    )(agg, cur_tab, ee, dinv32, wm, wu)


def _ro_body(cur_ref, wp_ref, wro_ref, bro_ref, out_ref):
    k = jnp.float32(NODES_PER_GRAPH)
    pooled = jnp.concatenate(
        [jnp.sum(cur_ref[0], axis=0, keepdims=True) / k,
         jnp.sum(cur_ref[1], axis=0, keepdims=True) / k], axis=1)
    fp = jnp.dot(pooled, wp_ref[...], preferred_element_type=_f32)
    feats = jnp.concatenate(
        [jnp.broadcast_to(_relu(fp), (NODES_PER_GRAPH, F)),
         _relu(jnp.concatenate([cur_ref[0], cur_ref[1]], axis=1))], axis=1)
    out_ref[...] = (
        jnp.dot(feats, wro_ref[...], preferred_element_type=_f32) + bro_ref[...])


def _ro(cur_tab, w_pool, w_ro, bro):
    return pl.pallas_call(
        _ro_body,
        grid=(NGRAPH,),
        in_specs=[
            pl.BlockSpec((2, NODES_PER_GRAPH, HALF), lambda i: (0, i, 0)),
            pl.BlockSpec((F, F), lambda i: (0, 0)),
            pl.BlockSpec((2 * F, 1), lambda i: (0, 0)),
            pl.BlockSpec((1, 1), lambda i: (0, 0)),
        ],
        out_specs=pl.BlockSpec((NODES_PER_GRAPH, 1), lambda i: (i, 0)),
        out_shape=jax.ShapeDtypeStruct((N, 1), _f32),
    )(cur_tab, w_pool, w_ro, bro)


def kernel(x, edge_index, batch, batch_size, W_init, W_e1, W_e2, W_msg,
           W_upd, W_pool, W_ro, b_ro):
    del batch, batch_size
    pad = EPAD - E
    rows = jnp.concatenate(
        [edge_index[0], jnp.zeros((pad,), jnp.int32)]).reshape(NTILE, NCH, CH)
    rows2 = jnp.concatenate([rows, rows + N], axis=0)
    cols = jnp.concatenate(
        [edge_index[1], jnp.full((pad,), TRASH, jnp.int32)]).reshape(NTILE, NCH, CH)
    zz = jnp.zeros((ROWS_PER_TILE, HALF), _f32)

    cur_tab, hp_tab = _p0(x, W_init, W_e1)

    aggh = _propagate(hp_tab.reshape(2 * N, HALF), rows2, cols, zz)
    ee, dinv32 = _p2(aggh, W_e2, _dmax(aggh))

    bro = b_ro.reshape(1, 1)

    for i in range(NLAYERS):
        agg = _propagate(cur_tab.reshape(2 * N, HALF), rows2, cols, zz)
        cur_tab = _p3(agg, cur_tab, ee, dinv32, W_msg[i], W_upd[i])

    return _ro(cur_tab, W_pool, W_ro, bro)
